# baseline (speedup 1.0000x reference)
import functools

import jax
import jax.numpy as jnp
from jax import lax
from jax.experimental import pallas as pl
from jax.experimental.pallas import tpu as pltpu
from jax.experimental.pallas import tpu_sc as plsc

VOCAB = 100000
EMBED_DIM = 128
CTX_LEN = 200

_CHUNK = 8
_NCHUNKS = CTX_LEN // _CHUNK

_mesh = plsc.VectorSubcoreMesh(core_axis_name="c", subcore_axis_name="s", num_cores=1)


@functools.partial(
    pl.kernel,
    mesh=_mesh,
    out_type=jax.ShapeDtypeStruct((CTX_LEN, EMBED_DIM), jnp.float32),
    scratch_types=[
        pltpu.VMEM((_CHUNK,), jnp.int32),
        pltpu.VMEM((_CHUNK, EMBED_DIM), jnp.float32),
        pltpu.SemaphoreType.DMA,
    ],
)
def _sc_gather(idx_hbm, table_hbm, out_hbm, idx_v, rows_v, sem):
    wid = lax.axis_index("s") * 2 + lax.axis_index("c")

    @pl.when(wid < 0)
    def _():
        base = wid * _CHUNK
        pltpu.sync_copy(idx_hbm.at[pl.ds(base, _CHUNK)], idx_v)
        pltpu.async_copy(table_hbm.at[idx_v], rows_v, sem).wait()
        pltpu.sync_copy(rows_v, out_hbm.at[pl.ds(base, _CHUNK)])


_NS = 1
_BV = 14336
_NPS = 7


def _tc_matvec_body(idx_sref, emb_ref, *refs):
    w_refs = refs[:_NS]
    b_refs = refs[_NS:2 * _NS]
    out_ref = refs[2 * _NS]
    mean_ref = refs[2 * _NS + 1]
    rows_ref = refs[2 * _NS + 2]
    sem = refs[2 * _NS + 3]

    @pl.when(pl.program_id(0) == 0)
    def _():
        def issue(j, carry):
            base = j * 8
            for u in range(8):
                pltpu.make_async_copy(
                    emb_ref.at[pl.ds(idx_sref[base + u], 1)],
                    rows_ref.at[pl.ds(base + u, 1)],
                    sem,
                ).start()
            return carry

        jax.lax.fori_loop(0, CTX_LEN // 8, issue, 0)
        pltpu.make_async_copy(
            emb_ref.at[pl.ds(0, CTX_LEN)], rows_ref, sem
        ).wait()
        m = jnp.sum(rows_ref[...], axis=0, keepdims=True) * (1.0 / CTX_LEN)
        mean_ref[...] = m

    m = mean_ref[...]
    accs = [
        jax.lax.dot_general(
            m, w_refs[s][...],
            (((1,), (1,)), ((), ())),
            preferred_element_type=jnp.float32,
        ) + b_refs[s][...]
        for s in range(_NS)
    ]
    out_ref[...] = jnp.concatenate(accs, axis=0)


def kernel(context_words, embeddings, W, b):
    b2d = b.reshape(1, VOCAB)
    w_specs = [
        pl.BlockSpec((_BV, EMBED_DIM), lambda i, idx, s=s: (s * _NPS + i, 0))
        for s in range(_NS)
    ]
    b_specs = [
        pl.BlockSpec((1, _BV), lambda i, idx, s=s: (0, s * _NPS + i))
        for s in range(_NS)
    ]
    grid_spec = pltpu.PrefetchScalarGridSpec(
        num_scalar_prefetch=1,
        grid=(_NPS,),
        in_specs=[pl.BlockSpec(memory_space=pltpu.MemorySpace.HBM)] + w_specs + b_specs,
        out_specs=pl.BlockSpec((_NS, _BV), lambda i, idx: (0, i)),
        scratch_shapes=[
            pltpu.VMEM((1, EMBED_DIM), jnp.float32),
            pltpu.VMEM((CTX_LEN, EMBED_DIM), jnp.float32),
            pltpu.SemaphoreType.DMA,
        ],
    )
    out = pl.pallas_call(
        _tc_matvec_body,
        grid_spec=grid_spec,
        out_shape=jax.ShapeDtypeStruct((_NS, _NPS * _BV), jnp.float32),
    )(context_words, embeddings, *([W] * _NS), *([b2d] * _NS))
    return out.reshape(_NS * _NPS * _BV)[:VOCAB]


# speedup vs baseline: 1.0167x; 1.0167x over previous
import functools

import jax
import jax.numpy as jnp
from jax import lax
from jax.experimental import pallas as pl
from jax.experimental.pallas import tpu as pltpu
from jax.experimental.pallas import tpu_sc as plsc

VOCAB = 100000
EMBED_DIM = 128
CTX_LEN = 200

_CHUNK = 8
_NCHUNKS = CTX_LEN // _CHUNK

_mesh = plsc.VectorSubcoreMesh(core_axis_name="c", subcore_axis_name="s", num_cores=1)


@functools.partial(
    pl.kernel,
    mesh=_mesh,
    out_type=jax.ShapeDtypeStruct((CTX_LEN, EMBED_DIM), jnp.float32),
    scratch_types=[
        pltpu.VMEM((_CHUNK,), jnp.int32),
        pltpu.VMEM((_CHUNK, EMBED_DIM), jnp.float32),
        pltpu.SemaphoreType.DMA,
    ],
)
def _sc_gather(idx_hbm, table_hbm, out_hbm, idx_v, rows_v, sem):
    wid = lax.axis_index("s") * 2 + lax.axis_index("c")

    @pl.when(wid < 0)
    def _():
        base = wid * _CHUNK
        pltpu.sync_copy(idx_hbm.at[pl.ds(base, _CHUNK)], idx_v)
        pltpu.async_copy(table_hbm.at[idx_v], rows_v, sem).wait()
        pltpu.sync_copy(rows_v, out_hbm.at[pl.ds(base, _CHUNK)])


_NS = 2
_BV = 8448
_NPS = 6


def _tc_matvec_body(idx_sref, emb_ref, *refs):
    w_refs = refs[:_NS]
    b_refs = refs[_NS:2 * _NS]
    out_ref = refs[2 * _NS]
    mean_ref = refs[2 * _NS + 1]
    rows_ref = refs[2 * _NS + 2]
    sem = refs[2 * _NS + 3]

    @pl.when(pl.program_id(0) == 0)
    def _():
        def issue(j, carry):
            base = j * 8
            for u in range(8):
                pltpu.make_async_copy(
                    emb_ref.at[pl.ds(idx_sref[base + u], 1)],
                    rows_ref.at[pl.ds(base + u, 1)],
                    sem,
                ).start()
            return carry

        jax.lax.fori_loop(0, CTX_LEN // 8, issue, 0)
        pltpu.make_async_copy(
            emb_ref.at[pl.ds(0, CTX_LEN)], rows_ref, sem
        ).wait()
        m = jnp.sum(rows_ref[...], axis=0, keepdims=True) * (1.0 / CTX_LEN)
        mean_ref[...] = m

    m = mean_ref[...]
    accs = [
        jax.lax.dot_general(
            m, w_refs[s][...],
            (((1,), (1,)), ((), ())),
            preferred_element_type=jnp.float32,
        ) + b_refs[s][...]
        for s in range(_NS)
    ]
    out_ref[...] = jnp.concatenate(accs, axis=0)


def kernel(context_words, embeddings, W, b):
    b2d = b.reshape(1, VOCAB)
    w_specs = [
        pl.BlockSpec((_BV, EMBED_DIM), lambda i, idx, s=s: (s * _NPS + i, 0))
        for s in range(_NS)
    ]
    b_specs = [
        pl.BlockSpec((1, _BV), lambda i, idx, s=s: (0, s * _NPS + i))
        for s in range(_NS)
    ]
    grid_spec = pltpu.PrefetchScalarGridSpec(
        num_scalar_prefetch=1,
        grid=(_NPS,),
        in_specs=[pl.BlockSpec(memory_space=pltpu.MemorySpace.HBM)] + w_specs + b_specs,
        out_specs=pl.BlockSpec((_NS, _BV), lambda i, idx: (0, i)),
        scratch_shapes=[
            pltpu.VMEM((1, EMBED_DIM), jnp.float32),
            pltpu.VMEM((CTX_LEN, EMBED_DIM), jnp.float32),
            pltpu.SemaphoreType.DMA,
        ],
    )
    out = pl.pallas_call(
        _tc_matvec_body,
        grid_spec=grid_spec,
        out_shape=jax.ShapeDtypeStruct((_NS, _NPS * _BV), jnp.float32),
    )(context_words, embeddings, *([W] * _NS), *([b2d] * _NS))
    return out.reshape(_NS * _NPS * _BV)[:VOCAB]


# speedup vs baseline: 1.0371x; 1.0201x over previous
import functools

import jax
import jax.numpy as jnp
from jax import lax
from jax.experimental import pallas as pl
from jax.experimental.pallas import tpu as pltpu
from jax.experimental.pallas import tpu_sc as plsc

VOCAB = 100000
EMBED_DIM = 128
CTX_LEN = 200

_CHUNK = 8
_NCHUNKS = CTX_LEN // _CHUNK

_mesh = plsc.VectorSubcoreMesh(core_axis_name="c", subcore_axis_name="s", num_cores=1)


@functools.partial(
    pl.kernel,
    mesh=_mesh,
    out_type=jax.ShapeDtypeStruct((CTX_LEN, EMBED_DIM), jnp.float32),
    scratch_types=[
        pltpu.VMEM((_CHUNK,), jnp.int32),
        pltpu.VMEM((_CHUNK, EMBED_DIM), jnp.float32),
        pltpu.SemaphoreType.DMA,
    ],
)
def _sc_gather(idx_hbm, table_hbm, out_hbm, idx_v, rows_v, sem):
    wid = lax.axis_index("s") * 2 + lax.axis_index("c")

    @pl.when(wid < 0)
    def _():
        base = wid * _CHUNK
        pltpu.sync_copy(idx_hbm.at[pl.ds(base, _CHUNK)], idx_v)
        pltpu.async_copy(table_hbm.at[idx_v], rows_v, sem).wait()
        pltpu.sync_copy(rows_v, out_hbm.at[pl.ds(base, _CHUNK)])


_NS = 2
_BV = 7168
_NPS = 7


def _tc_matvec_body(idx_sref, emb_ref, *refs):
    w_refs = refs[:_NS]
    b_refs = refs[_NS:2 * _NS]
    out_ref = refs[2 * _NS]
    mean_ref = refs[2 * _NS + 1]
    rows_ref = refs[2 * _NS + 2]
    sem = refs[2 * _NS + 3]

    @pl.when(pl.program_id(0) == 0)
    def _():
        def issue(j, carry):
            base = j * 8
            for u in range(8):
                pltpu.make_async_copy(
                    emb_ref.at[pl.ds(idx_sref[base + u], 1)],
                    rows_ref.at[pl.ds(base + u, 1)],
                    sem,
                ).start()
            return carry

        jax.lax.fori_loop(0, CTX_LEN // 8, issue, 0)
        pltpu.make_async_copy(
            emb_ref.at[pl.ds(0, CTX_LEN)], rows_ref, sem
        ).wait()
        m = jnp.sum(rows_ref[...], axis=0, keepdims=True) * (1.0 / CTX_LEN)
        mean_ref[...] = m

    m = mean_ref[...]
    accs = [
        jax.lax.dot_general(
            m, w_refs[s][...],
            (((1,), (1,)), ((), ())),
            preferred_element_type=jnp.float32,
        ) + b_refs[s][...]
        for s in range(_NS)
    ]
    out_ref[...] = jnp.concatenate(accs, axis=0)


def kernel(context_words, embeddings, W, b):
    b2d = b.reshape(1, VOCAB)
    w_specs = [
        pl.BlockSpec((_BV, EMBED_DIM), lambda i, idx, s=s: (s * _NPS + i, 0))
        for s in range(_NS)
    ]
    b_specs = [
        pl.BlockSpec((1, _BV), lambda i, idx, s=s: (0, s * _NPS + i))
        for s in range(_NS)
    ]
    grid_spec = pltpu.PrefetchScalarGridSpec(
        num_scalar_prefetch=1,
        grid=(_NPS,),
        in_specs=[pl.BlockSpec(memory_space=pltpu.MemorySpace.HBM)] + w_specs + b_specs,
        out_specs=pl.BlockSpec((_NS, _BV), lambda i, idx: (0, i)),
        scratch_shapes=[
            pltpu.VMEM((1, EMBED_DIM), jnp.float32),
            pltpu.VMEM((CTX_LEN, EMBED_DIM), jnp.float32),
            pltpu.SemaphoreType.DMA,
        ],
    )
    out = pl.pallas_call(
        _tc_matvec_body,
        grid_spec=grid_spec,
        out_shape=jax.ShapeDtypeStruct((_NS, _NPS * _BV), jnp.float32),
    )(context_words, embeddings, *([W] * _NS), *([b2d] * _NS))
    return out.reshape(_NS * _NPS * _BV)[:VOCAB]
